# baseline (speedup 1.0000x reference)
import functools

import jax
import jax.numpy as jnp
from jax import lax
from jax.experimental import pallas as pl
from jax.experimental.pallas import tpu as pltpu, tpu_sc as plsc

N = 10000
E = 320000
D = 128
NW = 32
R = 320
NP = NW * R
EPW = E // NW
CHA = 2000
NGA = CHA // 16
NCA = EPW // CHA
BUFW = 1024
BUFS = BUFW + 16
BCAP = 10 * BUFW
NEG = float("-inf")


def _mm_body(x_ref, wt_ref, wp_ref, b2_ref, xt_ref, xp_ref):
    xx = x_ref[...]
    dn = (((1,), (1,)), ((), ()))
    xt_ref[...] = lax.dot_general(xx, wt_ref[...], dn,
                                  preferred_element_type=jnp.float32)
    xp_ref[...] = lax.dot_general(xx, wp_ref[...], dn,
                                  preferred_element_type=jnp.float32) + b2_ref[...]


def _matmuls(x_pad, wt, wp, b2):
    blk = NP // 8
    return pl.pallas_call(
        _mm_body,
        grid=(8,),
        in_specs=[
            pl.BlockSpec((blk, D), lambda i: (i, 0)),
            pl.BlockSpec((D, D), lambda i: (0, 0)),
            pl.BlockSpec((D, D), lambda i: (0, 0)),
            pl.BlockSpec((1, D), lambda i: (0, 0)),
        ],
        out_specs=[
            pl.BlockSpec((blk, D), lambda i: (i, 0)),
            pl.BlockSpec((blk, D), lambda i: (i, 0)),
        ],
        out_shape=[
            jax.ShapeDtypeStruct((NP, D), jnp.float32),
            jax.ShapeDtypeStruct((NP, D), jnp.float32),
        ],
    )(x_pad, wt, wp, b2)


def _bucket_body(srcv, dstv, dvec, bp, bd, cnts,
                 dstc, srcc, dc, bufp, bufd, cvv, cnt_ref, fl_ref):
    wid = lax.axis_index("s") * 2 + lax.axis_index("c")
    lanes = lax.iota(jnp.int32, 16)
    zi = jnp.zeros((16,), jnp.int32)

    for o in range(NW):
        cnt_ref[o] = 0
        fl_ref[o] = 0
    def _init(i, _):
        bufp[pl.ds(i * 16, 16)] = zi
        return 0
    lax.fori_loop(0, NW * BUFS // 16, _init, 0)

    ebase0 = wid * EPW

    def chunk_body(c, _):
        eb = ebase0 + c * CHA
        pltpu.sync_copy(dstv.at[pl.ds(eb, CHA)], dstc)
        pltpu.sync_copy(srcv.at[pl.ds(eb, CHA)], srcc)
        pltpu.sync_copy(dvec.at[pl.ds(eb, CHA)], dc)

        def gbody(g, _):
            gs = pl.ds(g * 16, 16)
            dsts = dstc[gs]
            srcs = srcc[gs]
            dvs = dc[gs]
            owner = (dsts * 6554) >> 21
            packed = srcs * 1024 + (dsts - owner * R)
            for l in range(16):
                o = owner[l]
                cn = cnt_ref[o]
                base = o * BUFS + cn
                bufp[pl.ds(base, 16)] = jnp.full((16,), packed[l], jnp.int32)
                bufd[pl.ds(base, 16)] = jnp.full((16,), dvs[l], jnp.float32)
                cn = cn + 1
                cnt_ref[o] = cn

                @pl.when(cn == BUFW)
                def _():
                    fl = fl_ref[o]
                    hb = pl.ds((wid * NW + o) * BCAP + fl * BUFW, BUFW)
                    vb = pl.ds(o * BUFS, BUFW)
                    pltpu.sync_copy(bufp.at[vb], bp.at[hb])
                    pltpu.sync_copy(bufd.at[vb], bd.at[hb])
                    fl_ref[o] = fl + 1
                    cnt_ref[o] = 0
            return 0
        lax.fori_loop(0, NGA, gbody, 0)
        return 0
    lax.fori_loop(0, NCA, chunk_body, 0)

    clo = zi
    chi = zi
    for o in range(NW):
        cn = cnt_ref[o]
        fl = fl_ref[o]
        total = fl * BUFW + cn

        @pl.when(cn > 0)
        def _():
            hb = pl.ds((wid * NW + o) * BCAP + fl * BUFW, BUFW)
            vb = pl.ds(o * BUFS, BUFW)
            pltpu.sync_copy(bufp.at[vb], bp.at[hb])
            pltpu.sync_copy(bufd.at[vb], bd.at[hb])
        if o < 16:
            clo = jnp.where(lanes == o, total, clo)
        else:
            chi = jnp.where(lanes == o - 16, total, chi)
    cvv[pl.ds(0, 16)] = clo
    cvv[pl.ds(16, 16)] = chi
    pltpu.sync_copy(cvv, cnts.at[wid])


_bucketize = functools.partial(
    pl.kernel,
    mesh=plsc.VectorSubcoreMesh(core_axis_name="c", subcore_axis_name="s"),
    out_type=[
        jax.ShapeDtypeStruct((NW * NW * BCAP,), jnp.int32),
        jax.ShapeDtypeStruct((NW * NW * BCAP,), jnp.float32),
        jax.ShapeDtypeStruct((NW, NW), jnp.int32),
    ],
    scratch_types=[
        pltpu.VMEM((CHA,), jnp.int32),
        pltpu.VMEM((CHA,), jnp.int32),
        pltpu.VMEM((CHA,), jnp.float32),
        pltpu.VMEM((NW * BUFS,), jnp.int32),
        pltpu.VMEM((NW * BUFS,), jnp.float32),
        pltpu.VMEM((NW,), jnp.int32),
        pltpu.SMEM((NW,), jnp.int32),
        pltpu.SMEM((NW,), jnp.int32),
    ],
)(_bucket_body)


def _sc_body(xt, xp, xpad, bp, bd, cnts, out,
             m, rows, stp, std, sidx, cv, xc, pc, acc, sem):
    wid = lax.axis_index("s") * 2 + lax.axis_index("c")
    lo = wid * R
    cnt_nodes = jnp.minimum(R, N - lo)

    neg = jnp.full((16,), NEG, jnp.float32)
    zf = jnp.zeros((16,), jnp.float32)

    def _init_m(i, _):
        m[pl.ds(i * 16, 16)] = neg
        return 0
    lax.fori_loop(0, R * D // 16, _init_m, 0)
    for j in range(D // 16):
        acc[pl.ds(j * 16, 16)] = zf

    pltpu.sync_copy(cnts, cv.at[pl.ds(0, NW * NW)])

    def sbody(s, _):
        ec = cv[pl.ds(s * NW + wid, 16)][0]

        @pl.when(ec > 0)
        def _():
            sb = pl.ds((s * NW + wid) * BCAP, BCAP)
            pltpu.sync_copy(bp.at[sb], stp.at[pl.ds(0, BCAP)])
            pltpu.sync_copy(bd.at[sb], std.at[pl.ds(0, BCAP)])
            nb = (ec + 127) >> 7

            def bbody(k, _):
                kb = k * 128
                for j in range(8):
                    pv = stp[pl.ds(kb + j * 16, 16)]
                    sv = jnp.minimum(jnp.maximum(pv >> 10, 0), NP - 1)
                    sidx[pl.ds(j * 16, 16)] = sv
                pltpu.async_copy(xt.at[sidx], rows, sem).wait()
                ci = jnp.minimum(128, ec - kb)

                def ebody(i, _):
                    pv = stp[pl.ds(kb + i, 16)][0]
                    dl = pv & 1023
                    dvs = std[pl.ds(kb + i, 16)][0]
                    base = dl * D
                    for j in range(D // 16):
                        sl = pl.ds(base + j * 16, 16)
                        m[sl] = jnp.maximum(m[sl], dvs * rows[i, pl.ds(j * 16, 16)])
                    return 0
                lax.fori_loop(0, ci, ebody, 0)
                return 0
            lax.fori_loop(0, nb, bbody, 0)
        return 0
    lax.fori_loop(0, NW, sbody, 0)

    def fbody(c2, _):
        base = lo + c2 * 16
        pltpu.sync_copy(xpad.at[pl.ds(base, 16)], xc)
        pltpu.sync_copy(xp.at[pl.ds(base, 16)], pc)

        def rbody(r, _):
            n = c2 * 16 + r

            @pl.when(n < cnt_nodes)
            def _():
                mb = n * D
                for j in range(D // 16):
                    jl = pl.ds(j * 16, 16)
                    mv = m[pl.ds(mb + j * 16, 16)]
                    hv = jnp.where(mv > NEG, mv + pc[r, jl], xc[r, jl])
                    acc[jl] = acc[jl] + hv
            return 0
        lax.fori_loop(0, 16, rbody, 0)
        return 0
    lax.fori_loop(0, R // 16, fbody, 0)

    pltpu.sync_copy(acc, out.at[wid])


_sc_kernel = functools.partial(
    pl.kernel,
    mesh=plsc.VectorSubcoreMesh(core_axis_name="c", subcore_axis_name="s"),
    out_type=jax.ShapeDtypeStruct((NW, D), jnp.float32),
    scratch_types=[
        pltpu.VMEM((R * D,), jnp.float32),
        pltpu.VMEM((128, D), jnp.float32),
        pltpu.VMEM((BCAP + 16,), jnp.int32),
        pltpu.VMEM((BCAP + 16,), jnp.float32),
        pltpu.VMEM((128,), jnp.int32),
        pltpu.VMEM((NW * NW + 16,), jnp.int32),
        pltpu.VMEM((16, D), jnp.float32),
        pltpu.VMEM((16, D), jnp.float32),
        pltpu.VMEM((D,), jnp.float32),
        pltpu.SemaphoreType.DMA,
    ],
)(_sc_body)


def kernel(x, d, edge_index, W_theta, b_theta, W_phi, b_phi):
    src = edge_index[0]
    dst = edge_index[1]
    x_pad = jnp.zeros((NP, D), jnp.float32).at[:N].set(x)
    b2 = (b_theta + b_phi).reshape(1, D)
    XT, XPb = _matmuls(x_pad, W_theta, W_phi, b2)
    bp, bd, cnts = _bucketize(src, dst, d)
    partials = _sc_kernel(XT, XPb, x_pad, bp, bd, cnts.reshape(NW * NW))
    return jnp.sum(partials, axis=0, keepdims=True) * (1.0 / N)


# speedup vs baseline: 1.0178x; 1.0178x over previous
import functools

import jax
import jax.numpy as jnp
from jax import lax
from jax.experimental import pallas as pl
from jax.experimental.pallas import tpu as pltpu, tpu_sc as plsc

N = 10000
E = 320000
D = 128
NW = 32
R = 320
NP = NW * R
EPW = E // NW
CHA = 2000
NGA = CHA // 16
NCA = EPW // CHA
BUFW = 1024
BUFS = BUFW + 16
BCAP = 10 * BUFW
NEG = float("-inf")


def _mm_body(x_ref, wt_ref, wp_ref, b2_ref, xt_ref, xp_ref):
    xx = x_ref[...]
    dn = (((1,), (1,)), ((), ()))
    xt_ref[...] = lax.dot_general(xx, wt_ref[...], dn,
                                  preferred_element_type=jnp.float32)
    xp_ref[...] = lax.dot_general(xx, wp_ref[...], dn,
                                  preferred_element_type=jnp.float32) + b2_ref[...]


def _matmuls(x_pad, wt, wp, b2):
    blk = NP // 8
    return pl.pallas_call(
        _mm_body,
        grid=(8,),
        in_specs=[
            pl.BlockSpec((blk, D), lambda i: (i, 0)),
            pl.BlockSpec((D, D), lambda i: (0, 0)),
            pl.BlockSpec((D, D), lambda i: (0, 0)),
            pl.BlockSpec((1, D), lambda i: (0, 0)),
        ],
        out_specs=[
            pl.BlockSpec((blk, D), lambda i: (i, 0)),
            pl.BlockSpec((blk, D), lambda i: (i, 0)),
        ],
        out_shape=[
            jax.ShapeDtypeStruct((NP, D), jnp.float32),
            jax.ShapeDtypeStruct((NP, D), jnp.float32),
        ],
    )(x_pad, wt, wp, b2)


def _bucket_body(srcv, dstv, dvec, bp, bd, cnts,
                 dstc, srcc, dc, bufp, bufd, cvv, cnt_ref, fl_ref):
    wid = lax.axis_index("s") * 2 + lax.axis_index("c")
    lanes = lax.iota(jnp.int32, 16)
    zi = jnp.zeros((16,), jnp.int32)

    for o in range(NW):
        cnt_ref[o] = 0
        fl_ref[o] = 0
    def _init(i, _):
        bufp[pl.ds(i * 16, 16)] = zi
        return 0
    lax.fori_loop(0, NW * BUFS // 16, _init, 0)

    ebase0 = wid * EPW

    def chunk_body(c, _):
        eb = ebase0 + c * CHA
        pltpu.sync_copy(dstv.at[pl.ds(eb, CHA)], dstc)
        pltpu.sync_copy(srcv.at[pl.ds(eb, CHA)], srcc)
        pltpu.sync_copy(dvec.at[pl.ds(eb, CHA)], dc)

        def gbody(g, _):
            gs = pl.ds(g * 16, 16)
            dsts = dstc[gs]
            srcs = srcc[gs]
            dvs = dc[gs]
            owner = (dsts * 6554) >> 21
            packed = srcs * 1024 + (dsts - owner * R)
            for l in range(16):
                o = owner[l]
                cn = cnt_ref[o]
                base = o * BUFS + cn
                bufp[pl.ds(base, 16)] = jnp.full((16,), packed[l], jnp.int32)
                bufd[pl.ds(base, 16)] = jnp.full((16,), dvs[l], jnp.float32)
                cn = cn + 1
                cnt_ref[o] = cn

                @pl.when(cn == BUFW)
                def _():
                    fl = fl_ref[o]
                    hb = pl.ds((wid * NW + o) * BCAP + fl * BUFW, BUFW)
                    vb = pl.ds(o * BUFS, BUFW)
                    pltpu.sync_copy(bufp.at[vb], bp.at[hb])
                    pltpu.sync_copy(bufd.at[vb], bd.at[hb])
                    fl_ref[o] = fl + 1
                    cnt_ref[o] = 0
            return 0
        lax.fori_loop(0, NGA, gbody, 0)
        return 0
    lax.fori_loop(0, NCA, chunk_body, 0)

    clo = zi
    chi = zi
    for o in range(NW):
        cn = cnt_ref[o]
        fl = fl_ref[o]
        total = fl * BUFW + cn

        @pl.when(cn > 0)
        def _():
            hb = pl.ds((wid * NW + o) * BCAP + fl * BUFW, BUFW)
            vb = pl.ds(o * BUFS, BUFW)
            pltpu.sync_copy(bufp.at[vb], bp.at[hb])
            pltpu.sync_copy(bufd.at[vb], bd.at[hb])
        if o < 16:
            clo = jnp.where(lanes == o, total, clo)
        else:
            chi = jnp.where(lanes == o - 16, total, chi)
    cvv[pl.ds(0, 16)] = clo
    cvv[pl.ds(16, 16)] = chi
    pltpu.sync_copy(cvv, cnts.at[wid])


_bucketize = functools.partial(
    pl.kernel,
    mesh=plsc.VectorSubcoreMesh(core_axis_name="c", subcore_axis_name="s"),
    out_type=[
        jax.ShapeDtypeStruct((NW * NW * BCAP,), jnp.int32),
        jax.ShapeDtypeStruct((NW * NW * BCAP,), jnp.float32),
        jax.ShapeDtypeStruct((NW, NW), jnp.int32),
    ],
    scratch_types=[
        pltpu.VMEM((CHA,), jnp.int32),
        pltpu.VMEM((CHA,), jnp.int32),
        pltpu.VMEM((CHA,), jnp.float32),
        pltpu.VMEM((NW * BUFS,), jnp.int32),
        pltpu.VMEM((NW * BUFS,), jnp.float32),
        pltpu.VMEM((NW,), jnp.int32),
        pltpu.SMEM((NW,), jnp.int32),
        pltpu.SMEM((NW,), jnp.int32),
    ],
)(_bucket_body)


STCH = 2048


def _dg(v, idx):
    dn = lax.GatherDimensionNumbers(
        offset_dims=(), collapsed_slice_dims=(0,), start_index_map=(0,))
    return lax.gather(v, idx[:, None], dn, slice_sizes=(1,),
                      mode=lax.GatherScatterMode.PROMISE_IN_BOUNDS)


def _sc_body(xt, xp, xpad, bp, bd, cnts, out,
             m, rows0, rows1, stp, std, sidx0, sidx1, cv, xc, pc, acc, sem):
    wid = lax.axis_index("s") * 2 + lax.axis_index("c")
    lo = wid * R
    cnt_nodes = jnp.minimum(R, N - lo)

    neg = jnp.full((16,), NEG, jnp.float32)
    zf = jnp.zeros((16,), jnp.float32)
    zi = jnp.zeros((16,), jnp.int32)
    onei = zi + 1

    def _init_m(i, _):
        m[pl.ds(i * 16, 16)] = neg
        return 0
    lax.fori_loop(0, R * D // 16, _init_m, 0)
    for j in range(D // 16):
        acc[pl.ds(j * 16, 16)] = zf

    pltpu.sync_copy(cnts, cv.at[pl.ds(0, NW * NW)])

    def _unpack_fire(kb, sidx, rows):
        for j in range(8):
            pv = stp[pl.ds(kb + j * 16, 16)]
            sv = jnp.minimum(jnp.maximum(pv >> 10, 0), NP - 1)
            sidx[pl.ds(j * 16, 16)] = sv
        pltpu.async_copy(xt.at[sidx], rows, sem)

    def _process(kb, ci, rows):
        def epair(t, _):
            i = kb + t * 2
            pvv = stp[pl.ds(i, 16)]
            dvv = std[pl.ds(i, 16)]
            dl0 = (pvv[0] & 1023) * D
            dl1 = (pvv[1] & 1023) * D
            d0 = _dg(dvv, zi)
            d1 = _dg(dvv, onei)
            r = t * 2
            for j in range(D // 16):
                sl = pl.ds(dl0 + j * 16, 16)
                m[sl] = jnp.maximum(m[sl], d0 * rows[r, pl.ds(j * 16, 16)])
            for j in range(D // 16):
                sl = pl.ds(dl1 + j * 16, 16)
                m[sl] = jnp.maximum(m[sl], d1 * rows[r + 1, pl.ds(j * 16, 16)])
            return 0
        lax.fori_loop(0, ci >> 1, epair, 0)

        @pl.when((ci & 1) == 1)
        def _():
            i = kb + ci - 1
            pvv = stp[pl.ds(i, 16)]
            dvv = std[pl.ds(i, 16)]
            dl0 = (pvv[0] & 1023) * D
            d0 = _dg(dvv, zi)
            r = ci - 1
            for j in range(D // 16):
                sl = pl.ds(dl0 + j * 16, 16)
                m[sl] = jnp.maximum(m[sl], d0 * rows[r, pl.ds(j * 16, 16)])

    def sbody(s, _):
        ec = cv[pl.ds(s * NW + wid, 16)][0]

        @pl.when(ec > 0)
        def _():
            bbase = (s * NW + wid) * BCAP
            nst = (ec + STCH - 1) >> 11

            def stbody(q, _):
                qb = q * STCH
                sb = pl.ds(bbase + qb, STCH)
                pltpu.sync_copy(bp.at[sb], stp.at[pl.ds(0, STCH)])
                pltpu.sync_copy(bd.at[sb], std.at[pl.ds(0, STCH)])
                ecq = jnp.minimum(STCH, ec - qb)
                nb = (ecq + 127) >> 7
                _unpack_fire(0, sidx0, rows0)

                def bbody(k, _):
                    kb = k * 128

                    @pl.when(k + 1 < nb)
                    def _():
                        kb1 = kb + 128

                        @pl.when((k & 1) == 0)
                        def _():
                            _unpack_fire(kb1, sidx1, rows1)

                        @pl.when((k & 1) == 1)
                        def _():
                            _unpack_fire(kb1, sidx0, rows0)

                    ci = jnp.minimum(128, ecq - kb)

                    @pl.when((k & 1) == 0)
                    def _():
                        pltpu.make_async_copy(xt.at[sidx0], rows0, sem).wait()
                        _process(kb, ci, rows0)

                    @pl.when((k & 1) == 1)
                    def _():
                        pltpu.make_async_copy(xt.at[sidx1], rows1, sem).wait()
                        _process(kb, ci, rows1)
                    return 0
                lax.fori_loop(0, nb, bbody, 0)
                return 0
            lax.fori_loop(0, nst, stbody, 0)
        return 0
    lax.fori_loop(0, NW, sbody, 0)

    def fbody(c2, _):
        base = lo + c2 * 16
        pltpu.sync_copy(xpad.at[pl.ds(base, 16)], xc)
        pltpu.sync_copy(xp.at[pl.ds(base, 16)], pc)

        def rbody(r, _):
            n = c2 * 16 + r

            @pl.when(n < cnt_nodes)
            def _():
                mb = n * D
                for j in range(D // 16):
                    jl = pl.ds(j * 16, 16)
                    mv = m[pl.ds(mb + j * 16, 16)]
                    hv = jnp.where(mv > NEG, mv + pc[r, jl], xc[r, jl])
                    acc[jl] = acc[jl] + hv
            return 0
        lax.fori_loop(0, 16, rbody, 0)
        return 0
    lax.fori_loop(0, R // 16, fbody, 0)

    pltpu.sync_copy(acc, out.at[wid])


_sc_kernel = functools.partial(
    pl.kernel,
    mesh=plsc.VectorSubcoreMesh(core_axis_name="c", subcore_axis_name="s"),
    out_type=jax.ShapeDtypeStruct((NW, D), jnp.float32),
    scratch_types=[
        pltpu.VMEM((R * D,), jnp.float32),
        pltpu.VMEM((128, D), jnp.float32),
        pltpu.VMEM((128, D), jnp.float32),
        pltpu.VMEM((STCH + 16,), jnp.int32),
        pltpu.VMEM((STCH + 16,), jnp.float32),
        pltpu.VMEM((128,), jnp.int32),
        pltpu.VMEM((128,), jnp.int32),
        pltpu.VMEM((NW * NW + 16,), jnp.int32),
        pltpu.VMEM((16, D), jnp.float32),
        pltpu.VMEM((16, D), jnp.float32),
        pltpu.VMEM((D,), jnp.float32),
        pltpu.SemaphoreType.DMA,
    ],
)(_sc_body)


def kernel(x, d, edge_index, W_theta, b_theta, W_phi, b_phi):
    src = edge_index[0]
    dst = edge_index[1]
    x_pad = jnp.zeros((NP, D), jnp.float32).at[:N].set(x)
    b2 = (b_theta + b_phi).reshape(1, D)
    XT, XPb = _matmuls(x_pad, W_theta, W_phi, b2)
    bp, bd, cnts = _bucketize(src, dst, d)
    partials = _sc_kernel(XT, XPb, x_pad, bp, bd, cnts.reshape(NW * NW))
    return jnp.sum(partials, axis=0, keepdims=True) * (1.0 / N)


# speedup vs baseline: 2.3055x; 2.2651x over previous
import functools

import jax
import jax.numpy as jnp
from jax import lax
from jax.experimental import pallas as pl
from jax.experimental.pallas import tpu as pltpu, tpu_sc as plsc

N = 10000
E = 320000
D = 128
NW = 32
R = 320
NP = NW * R
EPW = E // NW
CHA = 2000
NGA = CHA // 16
NCA = EPW // CHA
BUFW = 1024
BUFS = BUFW + 16
BCAP = 10 * BUFW
NEG = float("-inf")


def _mm_body(x_ref, wt_ref, wp_ref, b2_ref, xt_ref, xp_ref, xo_ref):
    xx = x_ref[...]
    dn = (((1,), (1,)), ((), ()))
    xt_ref[...] = lax.dot_general(xx, wt_ref[...], dn,
                                  preferred_element_type=jnp.float32)
    xp_ref[...] = lax.dot_general(xx, wp_ref[...], dn,
                                  preferred_element_type=jnp.float32) + b2_ref[...]
    xo_ref[...] = xx


def _matmuls(x_pad, wt, wp, b2):
    blk = NP // 8
    return pl.pallas_call(
        _mm_body,
        grid=(8,),
        in_specs=[
            pl.BlockSpec((blk, D), lambda i: (i, 0)),
            pl.BlockSpec((D, D), lambda i: (0, 0)),
            pl.BlockSpec((D, D), lambda i: (0, 0)),
            pl.BlockSpec((1, D), lambda i: (0, 0)),
        ],
        out_specs=[
            pl.BlockSpec((blk, D), lambda i: (i, 0)),
            pl.BlockSpec((blk, D), lambda i: (i, 0)),
            pl.BlockSpec((blk, D), lambda i: (i, 0)),
        ],
        out_shape=[
            jax.ShapeDtypeStruct((NP, D), jnp.float32),
            jax.ShapeDtypeStruct((NP, D), jnp.float32),
            jax.ShapeDtypeStruct((NP, D), jnp.float32),
        ],
    )(x_pad, wt, wp, b2)


def _bucket_body(srcv, dstv, dvec, bp, bd, cnts,
                 dstc, srcc, dc, bufp, bufd, cvv, cnt_ref, fl_ref):
    wid = lax.axis_index("s") * 2 + lax.axis_index("c")
    lanes = lax.iota(jnp.int32, 16)
    zi = jnp.zeros((16,), jnp.int32)

    for o in range(NW):
        cnt_ref[o] = 0
        fl_ref[o] = 0
    def _init(i, _):
        bufp[pl.ds(i * 16, 16)] = zi
        return 0
    lax.fori_loop(0, NW * BUFS // 16, _init, 0)

    ebase0 = wid * EPW

    def chunk_body(c, _):
        eb = ebase0 + c * CHA
        pltpu.sync_copy(dstv.at[pl.ds(eb, CHA)], dstc)
        pltpu.sync_copy(srcv.at[pl.ds(eb, CHA)], srcc)
        pltpu.sync_copy(dvec.at[pl.ds(eb, CHA)], dc)

        def gbody(g, _):
            gs = pl.ds(g * 16, 16)
            dsts = dstc[gs]
            srcs = srcc[gs]
            dvs = dc[gs]
            owner = (dsts * 6554) >> 21
            packed = srcs * 1024 + (dsts - owner * R)
            for l in range(16):
                o = owner[l]
                cn = cnt_ref[o]
                base = o * BUFS + cn
                bufp[pl.ds(base, 16)] = jnp.full((16,), packed[l], jnp.int32)
                bufd[pl.ds(base, 16)] = jnp.full((16,), dvs[l], jnp.float32)
                cn = cn + 1
                cnt_ref[o] = cn

                @pl.when(cn == BUFW)
                def _():
                    fl = fl_ref[o]
                    hb = pl.ds((wid * NW + o) * BCAP + fl * BUFW, BUFW)
                    vb = pl.ds(o * BUFS, BUFW)
                    pltpu.sync_copy(bufp.at[vb], bp.at[hb])
                    pltpu.sync_copy(bufd.at[vb], bd.at[hb])
                    fl_ref[o] = fl + 1
                    cnt_ref[o] = 0
            return 0
        lax.fori_loop(0, NGA, gbody, 0)
        return 0
    lax.fori_loop(0, NCA, chunk_body, 0)

    clo = zi
    chi = zi
    for o in range(NW):
        cn = cnt_ref[o]
        fl = fl_ref[o]
        total = fl * BUFW + cn

        @pl.when(cn > 0)
        def _():
            hb = pl.ds((wid * NW + o) * BCAP + fl * BUFW, BUFW)
            vb = pl.ds(o * BUFS, BUFW)
            pltpu.sync_copy(bufp.at[vb], bp.at[hb])
            pltpu.sync_copy(bufd.at[vb], bd.at[hb])
        if o < 16:
            clo = jnp.where(lanes == o, total, clo)
        else:
            chi = jnp.where(lanes == o - 16, total, chi)
    cvv[pl.ds(0, 16)] = clo
    cvv[pl.ds(16, 16)] = chi
    pltpu.sync_copy(cvv, cnts.at[wid])


_bucketize = functools.partial(
    pl.kernel,
    mesh=plsc.VectorSubcoreMesh(core_axis_name="c", subcore_axis_name="s"),
    out_type=[
        jax.ShapeDtypeStruct((NW * NW * BCAP,), jnp.int32),
        jax.ShapeDtypeStruct((NW * NW * BCAP,), jnp.float32),
        jax.ShapeDtypeStruct((NW, NW), jnp.int32),
    ],
    scratch_types=[
        pltpu.VMEM((CHA,), jnp.int32),
        pltpu.VMEM((CHA,), jnp.int32),
        pltpu.VMEM((CHA,), jnp.float32),
        pltpu.VMEM((NW * BUFS,), jnp.int32),
        pltpu.VMEM((NW * BUFS,), jnp.float32),
        pltpu.VMEM((NW,), jnp.int32),
        pltpu.SMEM((NW,), jnp.int32),
        pltpu.SMEM((NW,), jnp.int32),
    ],
)(_bucket_body)


STCH = 2048
DH = D // 2
NP2 = NP // 2


def _dg(v, idx):
    dn = lax.GatherDimensionNumbers(
        offset_dims=(), collapsed_slice_dims=(0,), start_index_map=(0,))
    return lax.gather(v, idx[:, None], dn, slice_sizes=(1,),
                      mode=lax.GatherScatterMode.PROMISE_IN_BOUNDS)


def _sc_body(xt2, xp2, xpad2, bp, bd, cnts, out,
             m, rows0, rows1, stp, std, sidx0, sidx1, cv, xc, pc, acc,
             xtsh, sem):
    sid = lax.axis_index("s")
    wid = sid * 2 + lax.axis_index("c")
    lo = wid * R
    cnt_nodes = jnp.minimum(R, N - lo)

    neg = jnp.full((16,), NEG, jnp.float32)
    zf = jnp.zeros((16,), jnp.float32)
    zi = jnp.zeros((16,), jnp.int32)
    onei = zi + 1

    for j in range(D // 16):
        acc[pl.ds(j * 16, 16)] = zf
    pltpu.sync_copy(cnts, cv.at[pl.ds(0, NW * NW)])

    srows = NP2 // 16

    def _unpack_fire(kb, sidx, rows):
        for j in range(8):
            pv = stp[pl.ds(kb + j * 16, 16)]
            sv = jnp.minimum(jnp.maximum(pv >> 11, 0), NP2 - 1)
            sidx[pl.ds(j * 16, 16)] = sv
        pltpu.async_copy(xtsh.at[sidx], rows, sem)

    def _process(kb, ci, rows):
        def epair(t, _):
            i = kb + t * 2
            pvv = stp[pl.ds(i, 16)]
            dvv = std[pl.ds(i, 16)]
            p0 = pvv[0]
            p1 = pvv[1]
            dl0 = (p0 & 1023) * DH
            dl1 = (p1 & 1023) * DH
            q0 = ((p0 >> 10) & 1) * DH
            q1 = ((p1 >> 10) & 1) * DH
            d0 = _dg(dvv, zi)
            d1 = _dg(dvv, onei)
            r = t * 2
            for j in range(DH // 16):
                sl = pl.ds(dl0 + j * 16, 16)
                m[sl] = jnp.maximum(m[sl], d0 * rows[r, pl.ds(q0 + j * 16, 16)])
            for j in range(DH // 16):
                sl = pl.ds(dl1 + j * 16, 16)
                m[sl] = jnp.maximum(m[sl], d1 * rows[r + 1, pl.ds(q1 + j * 16, 16)])
            return 0
        lax.fori_loop(0, ci >> 1, epair, 0)

        @pl.when((ci & 1) == 1)
        def _():
            i = kb + ci - 1
            pvv = stp[pl.ds(i, 16)]
            dvv = std[pl.ds(i, 16)]
            p0 = pvv[0]
            dl0 = (p0 & 1023) * DH
            q0 = ((p0 >> 10) & 1) * DH
            d0 = _dg(dvv, zi)
            r = ci - 1
            for j in range(DH // 16):
                sl = pl.ds(dl0 + j * 16, 16)
                m[sl] = jnp.maximum(m[sl], d0 * rows[r, pl.ds(q0 + j * 16, 16)])

    for hp in range(2):
        pltpu.sync_copy(xt2.at[pl.ds(hp * NP2 + sid * srows, srows)],
                        xtsh.at[pl.ds(sid * srows, srows)])
        plsc.subcore_barrier()

        def _init_m(i, _):
            m[pl.ds(i * 16, 16)] = neg
            return 0
        lax.fori_loop(0, R * DH // 16, _init_m, 0)

        def sbody(s, _):
            ec = cv[pl.ds(s * NW + wid, 16)][0]

            @pl.when(ec > 0)
            def _():
                bbase = (s * NW + wid) * BCAP
                nst = (ec + STCH - 1) >> 11

                def stbody(q, _):
                    qb = q * STCH
                    sb = pl.ds(bbase + qb, STCH)
                    pltpu.sync_copy(bp.at[sb], stp.at[pl.ds(0, STCH)])
                    pltpu.sync_copy(bd.at[sb], std.at[pl.ds(0, STCH)])
                    ecq = jnp.minimum(STCH, ec - qb)
                    nb = (ecq + 127) >> 7
                    _unpack_fire(0, sidx0, rows0)

                    def bbody(k, _):
                        kb = k * 128

                        @pl.when(k + 1 < nb)
                        def _():
                            kb1 = kb + 128

                            @pl.when((k & 1) == 0)
                            def _():
                                _unpack_fire(kb1, sidx1, rows1)

                            @pl.when((k & 1) == 1)
                            def _():
                                _unpack_fire(kb1, sidx0, rows0)

                        ci = jnp.minimum(128, ecq - kb)

                        @pl.when((k & 1) == 0)
                        def _():
                            pltpu.make_async_copy(xtsh.at[sidx0], rows0, sem).wait()
                            _process(kb, ci, rows0)

                        @pl.when((k & 1) == 1)
                        def _():
                            pltpu.make_async_copy(xtsh.at[sidx1], rows1, sem).wait()
                            _process(kb, ci, rows1)
                        return 0
                    lax.fori_loop(0, nb, bbody, 0)
                    return 0
                lax.fori_loop(0, nst, stbody, 0)
            return 0
        lax.fori_loop(0, NW, sbody, 0)

        def fbody(c2, _):
            base = hp * NP + lo + c2 * 16
            pltpu.sync_copy(xpad2.at[pl.ds(base, 16)], xc)
            pltpu.sync_copy(xp2.at[pl.ds(base, 16)], pc)

            def rbody(r, _):
                n = c2 * 16 + r

                @pl.when(n < cnt_nodes)
                def _():
                    mb = n * DH
                    for j in range(DH // 16):
                        jl = pl.ds(j * 16, 16)
                        al = pl.ds(hp * DH + j * 16, 16)
                        mv = m[pl.ds(mb + j * 16, 16)]
                        hv = jnp.where(mv > NEG, mv + pc[r, jl], xc[r, jl])
                        acc[al] = acc[al] + hv
                return 0
            lax.fori_loop(0, 16, rbody, 0)
            return 0
        lax.fori_loop(0, R // 16, fbody, 0)

        if hp == 0:
            plsc.subcore_barrier()

    pltpu.sync_copy(acc, out.at[wid])


_sc_kernel = functools.partial(
    pl.kernel,
    mesh=plsc.VectorSubcoreMesh(core_axis_name="c", subcore_axis_name="s"),
    out_type=jax.ShapeDtypeStruct((NW, D), jnp.float32),
    scratch_types=[
        pltpu.VMEM((R * DH,), jnp.float32),
        pltpu.VMEM((128, D), jnp.float32),
        pltpu.VMEM((128, D), jnp.float32),
        pltpu.VMEM((STCH + 16,), jnp.int32),
        pltpu.VMEM((STCH + 16,), jnp.float32),
        pltpu.VMEM((128,), jnp.int32),
        pltpu.VMEM((128,), jnp.int32),
        pltpu.VMEM((NW * NW + 16,), jnp.int32),
        pltpu.VMEM((16, DH), jnp.float32),
        pltpu.VMEM((16, DH), jnp.float32),
        pltpu.VMEM((D,), jnp.float32),
        pltpu.VMEM_SHARED((NP2, D), jnp.float32),
        pltpu.SemaphoreType.DMA,
    ],
)(_sc_body)


def kernel(x, d, edge_index, W_theta, b_theta, W_phi, b_phi):
    src = edge_index[0]
    dst = edge_index[1]
    b2 = (b_theta + b_phi).reshape(1, D)
    XT, XPb, x_pad = _matmuls(x, W_theta, W_phi, b2)

    def halves(a):
        return a.reshape(NP, 2, DH).transpose(1, 0, 2).reshape(2 * NP, DH)

    XTpair = XT.reshape(NP2, 2, 2, DH).transpose(2, 0, 1, 3).reshape(NP, D)

    bp, bd, cnts = _bucketize(src, dst, d)
    partials = _sc_kernel(XTpair, halves(XPb), halves(x_pad),
                          bp, bd, cnts.reshape(NW * NW))
    return jnp.sum(partials, axis=0, keepdims=True) * (1.0 / N)


# speedup vs baseline: 2.3425x; 1.0161x over previous
import functools

import jax
import jax.numpy as jnp
from jax import lax
from jax.experimental import pallas as pl
from jax.experimental.pallas import tpu as pltpu, tpu_sc as plsc

N = 10000
E = 320000
D = 128
NW = 32
R = 320
NP = NW * R
EPW = E // NW
CHA = 2000
NGA = CHA // 16
NCA = EPW // CHA
BUFW = 1024
BUFS = BUFW + 16
BCAP = 10 * BUFW
NEG = float("-inf")


def _mm_body(x_ref, wt_ref, wp_ref, b2_ref, xt_ref, xp_ref, xo_ref):
    xx = x_ref[...]
    dn = (((1,), (1,)), ((), ()))
    xt_ref[...] = lax.dot_general(xx, wt_ref[...], dn,
                                  preferred_element_type=jnp.float32)
    xp_ref[...] = lax.dot_general(xx, wp_ref[...], dn,
                                  preferred_element_type=jnp.float32) + b2_ref[...]
    xo_ref[...] = xx


def _matmuls(x_pad, wt, wp, b2):
    blk = NP // 8
    return pl.pallas_call(
        _mm_body,
        grid=(8,),
        in_specs=[
            pl.BlockSpec((blk, D), lambda i: (i, 0)),
            pl.BlockSpec((D, D), lambda i: (0, 0)),
            pl.BlockSpec((D, D), lambda i: (0, 0)),
            pl.BlockSpec((1, D), lambda i: (0, 0)),
        ],
        out_specs=[
            pl.BlockSpec((blk, D), lambda i: (i, 0)),
            pl.BlockSpec((blk, D), lambda i: (i, 0)),
            pl.BlockSpec((blk, D), lambda i: (i, 0)),
        ],
        out_shape=[
            jax.ShapeDtypeStruct((NP, D), jnp.float32),
            jax.ShapeDtypeStruct((NP, D), jnp.float32),
            jax.ShapeDtypeStruct((NP, D), jnp.float32),
        ],
    )(x_pad, wt, wp, b2)


def _bucket_body(srcv, dstv, dvec, bp, bd, cnts,
                 dstc, srcc, dc, bufp, bufd, cvv, cnt_ref, fl_ref):
    wid = lax.axis_index("s") * 2 + lax.axis_index("c")
    lanes = lax.iota(jnp.int32, 16)
    zi = jnp.zeros((16,), jnp.int32)

    for o in range(NW):
        cnt_ref[o] = 0
        fl_ref[o] = 0
    def _init(i, _):
        bufp[pl.ds(i * 16, 16)] = zi
        return 0
    lax.fori_loop(0, NW * BUFS // 16, _init, 0)

    ebase0 = wid * EPW

    def chunk_body(c, _):
        eb = ebase0 + c * CHA
        pltpu.sync_copy(dstv.at[pl.ds(eb, CHA)], dstc)
        pltpu.sync_copy(srcv.at[pl.ds(eb, CHA)], srcc)
        pltpu.sync_copy(dvec.at[pl.ds(eb, CHA)], dc)

        def gbody(g, _):
            gs = pl.ds(g * 16, 16)
            dsts = dstc[gs]
            srcs = srcc[gs]
            dvs = dc[gs]
            owner = (dsts * 6554) >> 21
            packed = srcs * 1024 + (dsts - owner * R)
            for l in range(16):
                o = owner[l]
                cn = cnt_ref[o]
                base = o * BUFS + cn
                bufp[pl.ds(base, 16)] = jnp.full((16,), packed[l], jnp.int32)
                bufd[pl.ds(base, 16)] = jnp.full((16,), dvs[l], jnp.float32)
                cn = cn + 1
                cnt_ref[o] = cn

                @pl.when(cn == BUFW)
                def _():
                    fl = fl_ref[o]
                    hb = pl.ds((wid * NW + o) * BCAP + fl * BUFW, BUFW)
                    vb = pl.ds(o * BUFS, BUFW)
                    pltpu.sync_copy(bufp.at[vb], bp.at[hb])
                    pltpu.sync_copy(bufd.at[vb], bd.at[hb])
                    fl_ref[o] = fl + 1
                    cnt_ref[o] = 0
            return 0
        lax.fori_loop(0, NGA, gbody, 0)
        return 0
    lax.fori_loop(0, NCA, chunk_body, 0)

    clo = zi
    chi = zi
    for o in range(NW):
        cn = cnt_ref[o]
        fl = fl_ref[o]
        total = fl * BUFW + cn

        @pl.when(cn > 0)
        def _():
            hb = pl.ds((wid * NW + o) * BCAP + fl * BUFW, BUFW)
            vb = pl.ds(o * BUFS, BUFW)
            pltpu.sync_copy(bufp.at[vb], bp.at[hb])
            pltpu.sync_copy(bufd.at[vb], bd.at[hb])
        if o < 16:
            clo = jnp.where(lanes == o, total, clo)
        else:
            chi = jnp.where(lanes == o - 16, total, chi)
    cvv[pl.ds(0, 16)] = clo
    cvv[pl.ds(16, 16)] = chi
    pltpu.sync_copy(cvv, cnts.at[wid])


_bucketize = functools.partial(
    pl.kernel,
    mesh=plsc.VectorSubcoreMesh(core_axis_name="c", subcore_axis_name="s"),
    out_type=[
        jax.ShapeDtypeStruct((NW * NW * BCAP,), jnp.int32),
        jax.ShapeDtypeStruct((NW * NW * BCAP,), jnp.float32),
        jax.ShapeDtypeStruct((NW, NW), jnp.int32),
    ],
    scratch_types=[
        pltpu.VMEM((CHA,), jnp.int32),
        pltpu.VMEM((CHA,), jnp.int32),
        pltpu.VMEM((CHA,), jnp.float32),
        pltpu.VMEM((NW * BUFS,), jnp.int32),
        pltpu.VMEM((NW * BUFS,), jnp.float32),
        pltpu.VMEM((NW,), jnp.int32),
        pltpu.SMEM((NW,), jnp.int32),
        pltpu.SMEM((NW,), jnp.int32),
    ],
)(_bucket_body)


STCH = 2048
DH = D // 2
NP2 = NP // 2


def _dg(v, idx):
    dn = lax.GatherDimensionNumbers(
        offset_dims=(), collapsed_slice_dims=(0,), start_index_map=(0,))
    return lax.gather(v, idx[:, None], dn, slice_sizes=(1,),
                      mode=lax.GatherScatterMode.PROMISE_IN_BOUNDS)


def _sc_body(xt2, xp2, xpad2, bp, bd, cnts, out,
             m, rows0, rows1, rows2, rows3, stp, std,
             sidx0, sidx1, sidx2, sidx3, cv, xc, pc, acc,
             xtsh, sem):
    sid = lax.axis_index("s")
    wid = sid * 2 + lax.axis_index("c")
    lo = wid * R
    cnt_nodes = jnp.minimum(R, N - lo)

    neg = jnp.full((16,), NEG, jnp.float32)
    zf = jnp.zeros((16,), jnp.float32)
    zi = jnp.zeros((16,), jnp.int32)
    onei = zi + 1

    for j in range(D // 16):
        acc[pl.ds(j * 16, 16)] = zf
    pltpu.sync_copy(cnts, cv.at[pl.ds(0, NW * NW)])

    srows = NP2 // 16

    def _unpack_fire(kb, sidx, rows):
        for j in range(4):
            pv = stp[pl.ds(kb + j * 16, 16)]
            sv = jnp.minimum(jnp.maximum(pv >> 11, 0), NP2 - 1)
            sidx[pl.ds(j * 16, 16)] = sv
        pltpu.async_copy(xtsh.at[sidx], rows, sem)

    def _process(kb, ci, rows):
        def epair(t, _):
            i = kb + t * 2
            pvv = stp[pl.ds(i, 16)]
            dvv = std[pl.ds(i, 16)]
            p0 = pvv[0]
            p1 = pvv[1]
            dl0 = (p0 & 1023) * DH
            dl1 = (p1 & 1023) * DH
            q0 = ((p0 >> 10) & 1) * DH
            q1 = ((p1 >> 10) & 1) * DH
            d0 = _dg(dvv, zi)
            d1 = _dg(dvv, onei)
            r = t * 2
            for j in range(DH // 16):
                sl = pl.ds(dl0 + j * 16, 16)
                m[sl] = jnp.maximum(m[sl], d0 * rows[r, pl.ds(q0 + j * 16, 16)])
            for j in range(DH // 16):
                sl = pl.ds(dl1 + j * 16, 16)
                m[sl] = jnp.maximum(m[sl], d1 * rows[r + 1, pl.ds(q1 + j * 16, 16)])
            return 0
        lax.fori_loop(0, ci >> 1, epair, 0)

        @pl.when((ci & 1) == 1)
        def _():
            i = kb + ci - 1
            pvv = stp[pl.ds(i, 16)]
            dvv = std[pl.ds(i, 16)]
            p0 = pvv[0]
            dl0 = (p0 & 1023) * DH
            q0 = ((p0 >> 10) & 1) * DH
            d0 = _dg(dvv, zi)
            r = ci - 1
            for j in range(DH // 16):
                sl = pl.ds(dl0 + j * 16, 16)
                m[sl] = jnp.maximum(m[sl], d0 * rows[r, pl.ds(q0 + j * 16, 16)])

    for hp in range(2):
        pltpu.sync_copy(xt2.at[pl.ds(hp * NP2 + sid * srows, srows)],
                        xtsh.at[pl.ds(sid * srows, srows)])
        plsc.subcore_barrier()

        def _init_m(i, _):
            m[pl.ds(i * 16, 16)] = neg
            return 0
        lax.fori_loop(0, R * DH // 16, _init_m, 0)

        def sbody(s, _):
            ec = cv[pl.ds(s * NW + wid, 16)][0]

            @pl.when(ec > 0)
            def _():
                bbase = (s * NW + wid) * BCAP
                nst = (ec + STCH - 1) >> 11

                def stbody(q, _):
                    qb = q * STCH
                    sb = pl.ds(bbase + qb, STCH)
                    pltpu.sync_copy(bp.at[sb], stp.at[pl.ds(0, STCH)])
                    pltpu.sync_copy(bd.at[sb], std.at[pl.ds(0, STCH)])
                    ecq = jnp.minimum(STCH, ec - qb)
                    nb = (ecq + 63) >> 6
                    slots = ((sidx0, rows0), (sidx1, rows1),
                             (sidx2, rows2), (sidx3, rows3))
                    _unpack_fire(0, sidx0, rows0)

                    @pl.when(nb > 1)
                    def _():
                        _unpack_fire(64, sidx1, rows1)

                    @pl.when(nb > 2)
                    def _():
                        _unpack_fire(128, sidx2, rows2)

                    def bbody(k, _):
                        kb = k * 64

                        @pl.when(k + 3 < nb)
                        def _():
                            kb3 = kb + 192
                            for sl in range(4):
                                @pl.when((k & 3) == ((sl + 1) & 3))
                                def _(sl=sl):
                                    _unpack_fire(kb3, *slots[sl])

                        ci = jnp.minimum(64, ecq - kb)
                        for sl in range(4):
                            @pl.when((k & 3) == sl)
                            def _(sl=sl):
                                si, rw = slots[sl]
                                pltpu.make_async_copy(xtsh.at[si], rw, sem).wait()
                                _process(kb, ci, rw)
                        return 0
                    lax.fori_loop(0, nb, bbody, 0)
                    return 0
                lax.fori_loop(0, nst, stbody, 0)
            return 0
        lax.fori_loop(0, NW, sbody, 0)

        def fbody(c2, _):
            base = hp * NP + lo + c2 * 16
            pltpu.sync_copy(xpad2.at[pl.ds(base, 16)], xc)
            pltpu.sync_copy(xp2.at[pl.ds(base, 16)], pc)

            def rbody(r, _):
                n = c2 * 16 + r

                @pl.when(n < cnt_nodes)
                def _():
                    mb = n * DH
                    for j in range(DH // 16):
                        jl = pl.ds(j * 16, 16)
                        al = pl.ds(hp * DH + j * 16, 16)
                        mv = m[pl.ds(mb + j * 16, 16)]
                        hv = jnp.where(mv > NEG, mv + pc[r, jl], xc[r, jl])
                        acc[al] = acc[al] + hv
                return 0
            lax.fori_loop(0, 16, rbody, 0)
            return 0
        lax.fori_loop(0, R // 16, fbody, 0)

        if hp == 0:
            plsc.subcore_barrier()

    pltpu.sync_copy(acc, out.at[wid])


_sc_kernel = functools.partial(
    pl.kernel,
    mesh=plsc.VectorSubcoreMesh(core_axis_name="c", subcore_axis_name="s"),
    out_type=jax.ShapeDtypeStruct((NW, D), jnp.float32),
    scratch_types=[
        pltpu.VMEM((R * DH,), jnp.float32),
        pltpu.VMEM((64, D), jnp.float32),
        pltpu.VMEM((64, D), jnp.float32),
        pltpu.VMEM((64, D), jnp.float32),
        pltpu.VMEM((64, D), jnp.float32),
        pltpu.VMEM((STCH + 16,), jnp.int32),
        pltpu.VMEM((STCH + 16,), jnp.float32),
        pltpu.VMEM((64,), jnp.int32),
        pltpu.VMEM((64,), jnp.int32),
        pltpu.VMEM((64,), jnp.int32),
        pltpu.VMEM((64,), jnp.int32),
        pltpu.VMEM((NW * NW + 16,), jnp.int32),
        pltpu.VMEM((16, DH), jnp.float32),
        pltpu.VMEM((16, DH), jnp.float32),
        pltpu.VMEM((D,), jnp.float32),
        pltpu.VMEM_SHARED((NP2, D), jnp.float32),
        pltpu.SemaphoreType.DMA,
    ],
)(_sc_body)


def kernel(x, d, edge_index, W_theta, b_theta, W_phi, b_phi):
    src = edge_index[0]
    dst = edge_index[1]
    b2 = (b_theta + b_phi).reshape(1, D)
    XT, XPb, x_pad = _matmuls(x, W_theta, W_phi, b2)

    def halves(a):
        return a.reshape(NP, 2, DH).transpose(1, 0, 2).reshape(2 * NP, DH)

    XTpair = XT.reshape(NP2, 2, 2, DH).transpose(2, 0, 1, 3).reshape(NP, D)

    bp, bd, cnts = _bucketize(src, dst, d)
    partials = _sc_kernel(XTpair, halves(XPb), halves(x_pad),
                          bp, bd, cnts.reshape(NW * NW))
    return jnp.sum(partials, axis=0, keepdims=True) * (1.0 / N)
